# baseline (speedup 1.0000x reference)
import functools

import jax
import jax.numpy as jnp
from jax import lax
from jax.experimental import pallas as pl
from jax.experimental.pallas import tpu as pltpu
from jax.experimental.pallas import tpu_sc as plsc


def _scale_block(t_ref, o_ref):
    o_ref[...] = t_ref[...] * o_ref.shape[-1] ** 0.5


def _scale_table(table):
    v, d = table.shape
    block = 10000
    return pl.pallas_call(
        _scale_block,
        out_shape=jax.ShapeDtypeStruct((v, d), table.dtype),
        grid=(v // block,),
        in_specs=[pl.BlockSpec((block, d), lambda i: (i, 0))],
        out_specs=pl.BlockSpec((block, d), lambda i: (i, 0)),
    )(table)


@functools.cache
def _make_gather(B, D):
    num_cores, num_subcores = 2, 16
    nw = num_cores * num_subcores
    bpw = B // nw
    chunk = 128
    nchunk = bpw // chunk
    half = 2 * chunk
    nhalf = nchunk // 2
    assert nhalf % 3 == 1 and nhalf >= 4
    mesh = plsc.VectorSubcoreMesh(core_axis_name="c", subcore_axis_name="s")

    scratch = [pltpu.VMEM((nchunk, chunk), jnp.int32)]
    scratch += [pltpu.VMEM((half, D), jnp.float32) for _ in range(3)]
    scratch += [pltpu.SemaphoreType.DMA for _ in range(6)]

    @functools.partial(
        pl.kernel,
        mesh=mesh,
        out_type=jax.ShapeDtypeStruct((B, D), jnp.float32),
        scratch_types=scratch,
    )
    def gather(table_hbm, idx_hbm, out_hbm, idx_all, r0, r1, r2, g0, g1, g2, s0, s1, s2):
        rows = (r0, r1, r2)
        gsem = (g0, g1, g2)
        ssem = (s0, s1, s2)
        wid = lax.axis_index("s") * num_cores + lax.axis_index("c")
        cbase = wid * nchunk

        pltpu.sync_copy(idx_hbm.at[pl.ds(cbase, nchunk)], idx_all)

        def start_gathers(h, p):
            pltpu.async_copy(
                table_hbm.at[idx_all.at[2 * h]], rows[p].at[pl.ds(0, chunk)], gsem[p]
            )
            pltpu.async_copy(
                table_hbm.at[idx_all.at[2 * h + 1]],
                rows[p].at[pl.ds(chunk, chunk)],
                gsem[p],
            )

        def wait_gathers(p):
            pltpu.make_async_copy(
                table_hbm.at[idx_all.at[0]], rows[p], gsem[p]
            ).wait()

        def start_scatter(h, p):
            off = cbase * chunk + h * half
            pltpu.async_copy(rows[p], out_hbm.at[pl.ds(off, half)], ssem[p])

        def wait_scatter(h, p):
            off = cbase * chunk + h * half
            pltpu.make_async_copy(
                rows[p], out_hbm.at[pl.ds(off, half)], ssem[p]
            ).wait()

        start_gathers(0, 0)
        start_gathers(1, 1)
        start_gathers(2, 2)
        wait_gathers(0)
        start_scatter(0, 0)

        def body(g, carry):
            h1 = 3 * g + 1
            wait_gathers(1)
            start_scatter(h1, 1)
            wait_scatter(h1 - 1, 0)
            start_gathers(h1 + 2, 0)
            wait_gathers(2)
            start_scatter(h1 + 1, 2)
            wait_scatter(h1, 1)
            start_gathers(h1 + 3, 1)
            wait_gathers(0)
            start_scatter(h1 + 2, 0)
            wait_scatter(h1 + 1, 2)
            start_gathers(h1 + 4, 2)
            return carry

        lax.fori_loop(0, (nhalf - 4) // 3, body, 0)

        h = nhalf - 3
        wait_gathers(1)
        start_scatter(h, 1)
        wait_scatter(h - 1, 0)
        start_gathers(h + 2, 0)
        wait_gathers(2)
        start_scatter(h + 1, 2)
        wait_scatter(h, 1)
        wait_gathers(0)
        start_scatter(h + 2, 0)
        wait_scatter(h + 1, 2)
        wait_scatter(h + 2, 0)

    return gather


def kernel(x, table):
    n, s = x.shape
    d = table.shape[1]
    b = n * s
    scaled = _scale_table(table)
    idx2 = x.reshape(b // 128, 128).astype(jnp.int32)
    out = _make_gather(b, d)(scaled, idx2)
    return out.reshape(n, s, d)


# speedup vs baseline: 1.0035x; 1.0035x over previous
import functools

import jax
import jax.numpy as jnp
from jax import lax
from jax.experimental import pallas as pl
from jax.experimental.pallas import tpu as pltpu
from jax.experimental.pallas import tpu_sc as plsc


def _scale_block(t_ref, o_ref):
    o_ref[...] = t_ref[...] * o_ref.shape[-1] ** 0.5


def _scale_table(table):
    v, d = table.shape
    block = 20000
    return pl.pallas_call(
        _scale_block,
        out_shape=jax.ShapeDtypeStruct((v, d), table.dtype),
        grid=(v // block,),
        in_specs=[pl.BlockSpec((block, d), lambda i: (i, 0))],
        out_specs=pl.BlockSpec((block, d), lambda i: (i, 0)),
    )(table)


@functools.cache
def _make_gather(B, D):
    num_cores, num_subcores = 2, 16
    nw = num_cores * num_subcores
    bpw = B // nw
    chunk = 128
    nchunk = bpw // chunk
    half = 2 * chunk
    nhalf = nchunk // 2
    assert nhalf % 3 == 1 and nhalf >= 4
    mesh = plsc.VectorSubcoreMesh(core_axis_name="c", subcore_axis_name="s")

    scratch = [pltpu.VMEM((nchunk, chunk), jnp.int32)]
    scratch += [pltpu.VMEM((half, D), jnp.float32) for _ in range(3)]
    scratch += [pltpu.SemaphoreType.DMA for _ in range(6)]

    @functools.partial(
        pl.kernel,
        mesh=mesh,
        out_type=jax.ShapeDtypeStruct((B, D), jnp.float32),
        scratch_types=scratch,
    )
    def gather(table_hbm, idx_hbm, out_hbm, idx_all, r0, r1, r2, g0, g1, g2, s0, s1, s2):
        rows = (r0, r1, r2)
        gsem = (g0, g1, g2)
        ssem = (s0, s1, s2)
        wid = lax.axis_index("s") * num_cores + lax.axis_index("c")
        cbase = wid * nchunk

        pltpu.sync_copy(idx_hbm.at[pl.ds(cbase, nchunk)], idx_all)

        def start_gathers(h, p):
            pltpu.async_copy(
                table_hbm.at[idx_all.at[2 * h]], rows[p].at[pl.ds(0, chunk)], gsem[p]
            )
            pltpu.async_copy(
                table_hbm.at[idx_all.at[2 * h + 1]],
                rows[p].at[pl.ds(chunk, chunk)],
                gsem[p],
            )

        def wait_gathers(p):
            pltpu.make_async_copy(
                table_hbm.at[idx_all.at[0]], rows[p], gsem[p]
            ).wait()

        def start_scatter(h, p):
            off = cbase * chunk + h * half
            pltpu.async_copy(rows[p], out_hbm.at[pl.ds(off, half)], ssem[p])

        def wait_scatter(h, p):
            off = cbase * chunk + h * half
            pltpu.make_async_copy(
                rows[p], out_hbm.at[pl.ds(off, half)], ssem[p]
            ).wait()

        start_gathers(0, 0)
        start_gathers(1, 1)
        start_gathers(2, 2)
        wait_gathers(0)
        start_scatter(0, 0)

        def body(g, carry):
            h1 = 3 * g + 1
            wait_gathers(1)
            start_scatter(h1, 1)
            wait_scatter(h1 - 1, 0)
            start_gathers(h1 + 2, 0)
            wait_gathers(2)
            start_scatter(h1 + 1, 2)
            wait_scatter(h1, 1)
            start_gathers(h1 + 3, 1)
            wait_gathers(0)
            start_scatter(h1 + 2, 0)
            wait_scatter(h1 + 1, 2)
            start_gathers(h1 + 4, 2)
            return carry

        lax.fori_loop(0, (nhalf - 4) // 3, body, 0)

        h = nhalf - 3
        wait_gathers(1)
        start_scatter(h, 1)
        wait_scatter(h - 1, 0)
        start_gathers(h + 2, 0)
        wait_gathers(2)
        start_scatter(h + 1, 2)
        wait_scatter(h, 1)
        wait_gathers(0)
        start_scatter(h + 2, 0)
        wait_scatter(h + 1, 2)
        wait_scatter(h + 2, 0)

    return gather


def kernel(x, table):
    n, s = x.shape
    d = table.shape[1]
    b = n * s
    scaled = _scale_table(table)
    idx2 = x.reshape(b // 128, 128).astype(jnp.int32)
    out = _make_gather(b, d)(scaled, idx2)
    return out.reshape(n, s, d)
